# Initial kernel scaffold; baseline (speedup 1.0000x reference)
import functools

import jax
import jax.numpy as jnp
from jax import lax
from jax.experimental import pallas as pl
from jax.experimental.pallas import tpu as pltpu
from jax.experimental.pallas import tpu_sc as plsc

N = 10000
E = 320000
D = 128

NC = 2
NS = 16
NW = NC * NS

CH = 128
NCHUNKS = E // CH
ROWS_PER_TILE = N // NS
EDGES_PER_TILE_HIST = E // NW

_mesh = plsc.VectorSubcoreMesh(core_axis_name="c", subcore_axis_name="s")


def _hist_body(src_hbm, dst_hbm, dego_hbm, degi_hbm, idx_v, ho_v, hi_v, sem):
    cid = lax.axis_index("c")
    sid = lax.axis_index("s")
    wid = sid * NC + cid

    zeros16 = jnp.zeros((16,), jnp.float32)
    ones16 = jnp.ones((16,), jnp.float32)

    @pl.loop(0, N // 16)
    def _zero(i):
        ho_v[pl.ds(i * 16, 16)] = zeros16
        hi_v[pl.ds(i * 16, 16)] = zeros16

    base = wid * EDGES_PER_TILE_HIST

    pltpu.async_copy(src_hbm.at[pl.ds(base, EDGES_PER_TILE_HIST)], idx_v, sem).wait()

    @pl.loop(0, EDGES_PER_TILE_HIST // 16)
    def _accum_src(i):
        v = idx_v[pl.ds(i * 16, 16)]
        plsc.addupdate_scatter(ho_v, [v], ones16)

    pltpu.async_copy(dst_hbm.at[pl.ds(base, EDGES_PER_TILE_HIST)], idx_v, sem).wait()

    @pl.loop(0, EDGES_PER_TILE_HIST // 16)
    def _accum_dst(i):
        v = idx_v[pl.ds(i * 16, 16)]
        plsc.addupdate_scatter(hi_v, [v], ones16)

    pltpu.sync_copy(ho_v, dego_hbm.at[wid])
    pltpu.sync_copy(hi_v, degi_hbm.at[wid])


@jax.jit
def _sc_hist(src, dst):
    k = pl.kernel(
        _hist_body,
        out_type=(
            jax.ShapeDtypeStruct((NW, N), jnp.float32),
            jax.ShapeDtypeStruct((NW, N), jnp.float32),
        ),
        mesh=_mesh,
        scratch_types=[
            pltpu.VMEM((EDGES_PER_TILE_HIST,), jnp.int32),
            pltpu.VMEM((N,), jnp.float32),
            pltpu.VMEM((N,), jnp.float32),
            pltpu.SemaphoreType.DMA,
        ],
    )
    return k(src, dst)


def _agg_body(h_hbm, src_hbm, dst_hbm, out_hbm,
              srcb, dstb, rows, zbuf, agg_sh, sem):
    cid = lax.axis_index("c")
    sid = lax.axis_index("s")
    wid = sid * NC + cid

    zeros16 = jnp.zeros((16,), jnp.float32)

    @pl.loop(0, 64)
    def _zero(r):
        for k8 in range(D // 16):
            zbuf[r, pl.ds(k8 * 16, 16)] = zeros16

    row0 = sid * ROWS_PER_TILE

    @pl.loop(0, ROWS_PER_TILE // 64)
    def _zinit(j):
        pltpu.sync_copy(zbuf, agg_sh.at[pl.ds(row0 + j * 64, 64)])

    rem = ROWS_PER_TILE % 64
    if rem:
        pltpu.sync_copy(zbuf.at[pl.ds(0, rem)],
                        agg_sh.at[pl.ds(row0 + (ROWS_PER_TILE // 64) * 64, rem)])

    plsc.subcore_barrier()

    nbase = NCHUNKS // NW
    nextra = NCHUNKS % NW
    start = wid * nbase + jnp.minimum(wid, nextra)
    cnt = jnp.where(wid < nextra, nbase + 1, nbase)

    @pl.loop(0, cnt)
    def _edges(i):
        c = start + i
        pltpu.async_copy(src_hbm.at[pl.ds(c * CH, CH)], srcb, sem).wait()
        pltpu.async_copy(dst_hbm.at[pl.ds(c * CH, CH)], dstb, sem).wait()
        pltpu.async_copy(h_hbm.at[srcb], rows, sem).wait()
        pltpu.sync_copy(rows, agg_sh.at[dstb], add=True)

    plsc.subcore_barrier()

    pltpu.sync_copy(agg_sh.at[pl.ds(row0, ROWS_PER_TILE)],
                    out_hbm.at[cid, pl.ds(row0, ROWS_PER_TILE)])


@jax.jit
def _sc_agg(h, src, dst):
    k = pl.kernel(
        _agg_body,
        out_type=jax.ShapeDtypeStruct((NC, N, D), jnp.float32),
        mesh=_mesh,
        scratch_types=[
            pltpu.VMEM((CH,), jnp.int32),
            pltpu.VMEM((CH,), jnp.int32),
            pltpu.VMEM((CH, D), jnp.float32),
            pltpu.VMEM((64, D), jnp.float32),
            pltpu.VMEM_SHARED((N, D), jnp.float32),
            pltpu.SemaphoreType.DMA,
        ],
    )
    return k(h, src, dst)


BLK = 400


def _mm_body(x_ref, w_ref, degp_ref, o_ref):
    deg = jnp.sum(degp_ref[...], axis=0)
    norm = lax.rsqrt(jnp.maximum(deg, 1.0))
    z = jnp.dot(x_ref[...], w_ref[...],
                preferred_element_type=jnp.float32,
                precision=lax.Precision.HIGHEST)
    o_ref[...] = z * norm[:, None]


@jax.jit
def _tc_matmul_scale(x, w, degp):
    return pl.pallas_call(
        _mm_body,
        out_shape=jax.ShapeDtypeStruct((N, D), jnp.float32),
        grid=(N // BLK,),
        in_specs=[
            pl.BlockSpec((BLK, D), lambda i: (i, 0)),
            pl.BlockSpec((D, D), lambda i: (0, 0)),
            pl.BlockSpec((NW, BLK), lambda i: (0, i)),
        ],
        out_specs=pl.BlockSpec((BLK, D), lambda i: (i, 0)),
    )(x, w, degp)


def _post_body(p_ref, degp_ref, b_ref, o_ref):
    deg = jnp.sum(degp_ref[...], axis=0)
    norm = lax.rsqrt(jnp.maximum(deg, 1.0))
    agg = p_ref[0] + p_ref[1]
    o_ref[...] = agg * norm[:, None] + b_ref[...][None, :]


@jax.jit
def _tc_post(p, degp, b):
    return pl.pallas_call(
        _post_body,
        out_shape=jax.ShapeDtypeStruct((N, D), jnp.float32),
        grid=(N // BLK,),
        in_specs=[
            pl.BlockSpec((NC, BLK, D), lambda i: (0, i, 0)),
            pl.BlockSpec((NW, BLK), lambda i: (0, i)),
            pl.BlockSpec((D,), lambda i: (0,)),
        ],
        out_specs=pl.BlockSpec((BLK, D), lambda i: (i, 0)),
    )(p, degp, b)


def kernel(ndata, edge_index, W, b):
    src = edge_index[0].astype(jnp.int32)
    dst = edge_index[1].astype(jnp.int32)
    dego_p, degi_p = _sc_hist(src, dst)
    h = _tc_matmul_scale(ndata, W, dego_p)
    p = _sc_agg(h, src, dst)
    return _tc_post(p, degi_p, b)


# speedup vs baseline: 7.5651x; 7.5651x over previous
import dataclasses

import jax
import jax.numpy as jnp
from jax import lax
from jax.experimental import pallas as pl
from jax.experimental.pallas import tpu as pltpu
from jax.experimental.pallas import tpu_sc as plsc

N = 10000
NPAD = 10240
E = 320000
D = 128

NC = 2
NS = 16
NW = NC * NS

CH = 128
NCHUNKS = E // CH
ROWS_PER_TILE = NPAD // NS
EDGES_PER_TILE_HIST = E // NW

_mesh = plsc.VectorSubcoreMesh(core_axis_name="c", subcore_axis_name="s")

_sc_params = pltpu.CompilerParams()
if "needs_layout_passes" in pltpu.CompilerParams.__dataclass_fields__:
    _sc_params = dataclasses.replace(_sc_params, needs_layout_passes=False)


def _hist_body(src_hbm, dst_hbm, dego_hbm, degi_hbm, idx_v, ho_v, hi_v, sem):
    cid = lax.axis_index("c")
    sid = lax.axis_index("s")
    wid = sid * NC + cid

    zeros16 = jnp.zeros((16,), jnp.float32)
    ones16 = jnp.ones((16,), jnp.float32)

    @pl.loop(0, NPAD // 16)
    def _zero(i):
        ho_v[pl.ds(i * 16, 16)] = zeros16
        hi_v[pl.ds(i * 16, 16)] = zeros16

    base = wid * EDGES_PER_TILE_HIST

    pltpu.async_copy(src_hbm.at[pl.ds(base, EDGES_PER_TILE_HIST)], idx_v, sem).wait()

    @pl.loop(0, EDGES_PER_TILE_HIST // 16)
    def _accum_src(i):
        v = idx_v[pl.ds(i * 16, 16)]
        plsc.addupdate_scatter(ho_v, [v], ones16)

    pltpu.async_copy(dst_hbm.at[pl.ds(base, EDGES_PER_TILE_HIST)], idx_v, sem).wait()

    @pl.loop(0, EDGES_PER_TILE_HIST // 16)
    def _accum_dst(i):
        v = idx_v[pl.ds(i * 16, 16)]
        plsc.addupdate_scatter(hi_v, [v], ones16)

    pltpu.sync_copy(ho_v, dego_hbm.at[wid])
    pltpu.sync_copy(hi_v, degi_hbm.at[wid])


@jax.jit
def _sc_hist(src, dst):
    k = pl.kernel(
        _hist_body,
        out_type=(
            jax.ShapeDtypeStruct((NW, NPAD), jnp.float32),
            jax.ShapeDtypeStruct((NW, NPAD), jnp.float32),
        ),
        mesh=_mesh,
        scratch_types=[
            pltpu.VMEM((EDGES_PER_TILE_HIST,), jnp.int32),
            pltpu.VMEM((NPAD,), jnp.float32),
            pltpu.VMEM((NPAD,), jnp.float32),
            pltpu.SemaphoreType.DMA,
        ],
        compiler_params=_sc_params,
    )
    return k(src, dst)


def _agg_body(h_hbm, src_hbm, dst_hbm, out_hbm,
              srcb, dstb, rows, zbuf, agg_sh, sem):
    cid = lax.axis_index("c")
    sid = lax.axis_index("s")
    wid = sid * NC + cid

    zeros16 = jnp.zeros((16,), jnp.float32)

    @pl.loop(0, 64)
    def _zero(r):
        for k8 in range(D // 16):
            zbuf[r, pl.ds(k8 * 16, 16)] = zeros16

    row0 = sid * ROWS_PER_TILE

    @pl.loop(0, ROWS_PER_TILE // 64)
    def _zinit(j):
        pltpu.sync_copy(zbuf, agg_sh.at[pl.ds(row0 + j * 64, 64)])

    plsc.subcore_barrier()

    nbase = NCHUNKS // NW
    nextra = NCHUNKS % NW
    start = wid * nbase + jnp.minimum(wid, nextra)
    cnt = jnp.where(wid < nextra, nbase + 1, nbase)

    @pl.loop(0, cnt)
    def _edges(i):
        c = start + i
        pltpu.async_copy(src_hbm.at[pl.ds(c * CH, CH)], srcb, sem).wait()
        pltpu.async_copy(dst_hbm.at[pl.ds(c * CH, CH)], dstb, sem).wait()
        pltpu.async_copy(h_hbm.at[srcb], rows, sem).wait()
        pltpu.sync_copy(rows, agg_sh.at[dstb], add=True)

    plsc.subcore_barrier()

    pltpu.sync_copy(agg_sh.at[pl.ds(row0, ROWS_PER_TILE)],
                    out_hbm.at[cid, pl.ds(row0, ROWS_PER_TILE)])


@jax.jit
def _sc_agg(h, src, dst):
    k = pl.kernel(
        _agg_body,
        out_type=jax.ShapeDtypeStruct((NC, NPAD, D), jnp.float32),
        mesh=_mesh,
        scratch_types=[
            pltpu.VMEM((CH,), jnp.int32),
            pltpu.VMEM((CH,), jnp.int32),
            pltpu.VMEM((CH, D), jnp.float32),
            pltpu.VMEM((64, D), jnp.float32),
            pltpu.VMEM_SHARED((NPAD, D), jnp.float32),
            pltpu.SemaphoreType.DMA,
        ],
        compiler_params=_sc_params,
    )
    return k(h, src, dst)


BLK = 512


def _mm_body(x_ref, w_ref, degp_ref, o_ref):
    deg = jnp.sum(degp_ref[...], axis=0)
    norm = lax.rsqrt(jnp.maximum(deg, 1.0))
    z = jnp.dot(x_ref[...], w_ref[...],
                preferred_element_type=jnp.float32,
                precision=lax.Precision.HIGHEST)
    o_ref[...] = z * norm[:, None]


@jax.jit
def _tc_matmul_scale(x, w, degp):
    return pl.pallas_call(
        _mm_body,
        out_shape=jax.ShapeDtypeStruct((NPAD, D), jnp.float32),
        grid=(NPAD // BLK,),
        in_specs=[
            pl.BlockSpec((BLK, D), lambda i: (i, 0)),
            pl.BlockSpec((D, D), lambda i: (0, 0)),
            pl.BlockSpec((NW, BLK), lambda i: (0, i)),
        ],
        out_specs=pl.BlockSpec((BLK, D), lambda i: (i, 0)),
    )(x, w, degp)


def _post_body(p_ref, degp_ref, b_ref, o_ref):
    deg = jnp.sum(degp_ref[...], axis=0)
    norm = lax.rsqrt(jnp.maximum(deg, 1.0))
    agg = p_ref[0] + p_ref[1]
    o_ref[...] = agg * norm[:, None] + b_ref[...][None, :]


@jax.jit
def _tc_post(p, degp, b):
    return pl.pallas_call(
        _post_body,
        out_shape=jax.ShapeDtypeStruct((NPAD, D), jnp.float32),
        grid=(NPAD // BLK,),
        in_specs=[
            pl.BlockSpec((NC, BLK, D), lambda i: (0, i, 0)),
            pl.BlockSpec((NW, BLK), lambda i: (0, i)),
            pl.BlockSpec((D,), lambda i: (0,)),
        ],
        out_specs=pl.BlockSpec((BLK, D), lambda i: (i, 0)),
    )(p, degp, b)


def kernel(ndata, edge_index, W, b):
    src = edge_index[0].astype(jnp.int32)
    dst = edge_index[1].astype(jnp.int32)
    x_pad = jnp.pad(ndata, ((0, NPAD - N), (0, 0)))
    dego_p, degi_p = _sc_hist(src, dst)
    h = _tc_matmul_scale(x_pad, W, dego_p)
    p = _sc_agg(h, src, dst)
    return _tc_post(p, degi_p, b)[:N]


# speedup vs baseline: 10.8801x; 1.4382x over previous
import dataclasses

import jax
import jax.numpy as jnp
from jax import lax
from jax.experimental import pallas as pl
from jax.experimental.pallas import tpu as pltpu
from jax.experimental.pallas import tpu_sc as plsc

N = 10000
NPAD = 10240
E = 320000
D = 128

NC = 2
NS = 16
NW = NC * NS

CH = 128
E_PAD = 327680
NCHP = E_PAD // CH
CPT = NCHP // NW
NBUF = 2
ROWS_PER_TILE = NPAD // NS
EDGES_PER_TILE_HIST = E // NW

_mesh = plsc.VectorSubcoreMesh(core_axis_name="c", subcore_axis_name="s")

_sc_params = pltpu.CompilerParams()
if "needs_layout_passes" in pltpu.CompilerParams.__dataclass_fields__:
    _sc_params = dataclasses.replace(_sc_params, needs_layout_passes=False)


def _hist_body(src_hbm, dst_hbm, dego_hbm, degi_hbm, idx_v, ho_v, hi_v, sem):
    cid = lax.axis_index("c")
    sid = lax.axis_index("s")
    wid = sid * NC + cid

    zeros16 = jnp.zeros((16,), jnp.float32)
    ones16 = jnp.ones((16,), jnp.float32)

    @pl.loop(0, NPAD // 16)
    def _zero(i):
        ho_v[pl.ds(i * 16, 16)] = zeros16
        hi_v[pl.ds(i * 16, 16)] = zeros16

    base = wid * EDGES_PER_TILE_HIST

    pltpu.async_copy(src_hbm.at[pl.ds(base, EDGES_PER_TILE_HIST)], idx_v, sem).wait()

    @pl.loop(0, EDGES_PER_TILE_HIST // 16)
    def _accum_src(i):
        v = idx_v[pl.ds(i * 16, 16)]
        plsc.addupdate_scatter(ho_v, [v], ones16)

    pltpu.async_copy(dst_hbm.at[pl.ds(base, EDGES_PER_TILE_HIST)], idx_v, sem).wait()

    @pl.loop(0, EDGES_PER_TILE_HIST // 16)
    def _accum_dst(i):
        v = idx_v[pl.ds(i * 16, 16)]
        plsc.addupdate_scatter(hi_v, [v], ones16)

    pltpu.sync_copy(ho_v, dego_hbm.at[wid])
    pltpu.sync_copy(hi_v, degi_hbm.at[wid])


@jax.jit
def _sc_hist(src, dst):
    k = pl.kernel(
        _hist_body,
        out_type=(
            jax.ShapeDtypeStruct((NW, NPAD), jnp.float32),
            jax.ShapeDtypeStruct((NW, NPAD), jnp.float32),
        ),
        mesh=_mesh,
        scratch_types=[
            pltpu.VMEM((EDGES_PER_TILE_HIST,), jnp.int32),
            pltpu.VMEM((NPAD,), jnp.float32),
            pltpu.VMEM((NPAD,), jnp.float32),
            pltpu.SemaphoreType.DMA,
        ],
        compiler_params=_sc_params,
    )
    return k(src, dst)


def _agg_body(h_hbm, pk_hbm, out_hbm,
              pkb, r0, r1, su0, su1, du0, du1, agg_sh,
              g0, g1, s0, s1):
    cid = lax.axis_index("c")
    sid = lax.axis_index("s")
    wid = sid * NC + cid

    rows = (r0, r1)
    src_u = (su0, su1)
    dst_u = (du0, du1)
    gsem = (g0, g1)
    ssem = (s0, s1)

    zeros16 = jnp.zeros((16,), jnp.float32)
    mask16 = jnp.full((16,), 0xFFFF, jnp.int32)
    sixteen = jnp.full((16,), 16, jnp.int32)

    def unpack(c, p):
        for k8 in range(CH // 16):
            v = pkb[c, pl.ds(k8 * 16, 16)]
            src_u[p][pl.ds(k8 * 16, 16)] = jnp.bitwise_and(v, mask16)
            dst_u[p][pl.ds(k8 * 16, 16)] = lax.shift_right_logical(v, sixteen)

    base = wid * CPT
    pltpu.sync_copy(pk_hbm.at[pl.ds(base, CPT)], pkb)

    @pl.loop(0, CH)
    def _zero(r):
        for k8 in range(D // 16):
            r0[r, pl.ds(k8 * 16, 16)] = zeros16

    row0 = sid * ROWS_PER_TILE

    @pl.loop(0, ROWS_PER_TILE // CH)
    def _zinit(j):
        pltpu.sync_copy(r0, agg_sh.at[pl.ds(row0 + j * CH, CH)])

    for p in range(NBUF):
        unpack(p, p)
        pltpu.async_copy(h_hbm.at[src_u[p]], rows[p], gsem[p])

    plsc.subcore_barrier()

    @pl.loop(0, CPT // 2 - 1)
    def _unit(j):
        u = j * 2
        for p in range(NBUF):
            pltpu.make_async_copy(h_hbm.at[src_u[p]], rows[p], gsem[p]).wait()
            pltpu.async_copy(rows[p], agg_sh.at[dst_u[p]], ssem[p], add=True)
        for p in range(NBUF):
            c = u + p
            pltpu.make_async_copy(rows[p], agg_sh.at[dst_u[p]], ssem[p]).wait()
            unpack(c + NBUF, p)
            pltpu.async_copy(h_hbm.at[src_u[p]], rows[p], gsem[p])

    for p in range(NBUF):
        pltpu.make_async_copy(h_hbm.at[src_u[p]], rows[p], gsem[p]).wait()
        pltpu.async_copy(rows[p], agg_sh.at[dst_u[p]], ssem[p], add=True)
    for p in range(NBUF):
        pltpu.make_async_copy(rows[p], agg_sh.at[dst_u[p]], ssem[p]).wait()

    plsc.subcore_barrier()

    pltpu.sync_copy(agg_sh.at[pl.ds(row0, ROWS_PER_TILE)],
                    out_hbm.at[cid, pl.ds(row0, ROWS_PER_TILE)])


@jax.jit
def _sc_agg(h, packed2d):
    k = pl.kernel(
        _agg_body,
        out_type=jax.ShapeDtypeStruct((NC, NPAD, D), jnp.float32),
        mesh=_mesh,
        scratch_types=[
            pltpu.VMEM((CPT, CH), jnp.int32),
            pltpu.VMEM((CH, D), jnp.float32),
            pltpu.VMEM((CH, D), jnp.float32),
            pltpu.VMEM((CH,), jnp.int32),
            pltpu.VMEM((CH,), jnp.int32),
            pltpu.VMEM((CH,), jnp.int32),
            pltpu.VMEM((CH,), jnp.int32),
            pltpu.VMEM_SHARED((NPAD, D), jnp.float32),
            pltpu.SemaphoreType.DMA,
            pltpu.SemaphoreType.DMA,
            pltpu.SemaphoreType.DMA,
            pltpu.SemaphoreType.DMA,
        ],
        compiler_params=_sc_params,
    )
    return k(h, packed2d)


BLK = 512


def _mm_body(x_ref, w_ref, degp_ref, o_ref):
    deg = jnp.sum(degp_ref[...], axis=0)
    norm = lax.rsqrt(jnp.maximum(deg, 1.0))
    z = jnp.dot(x_ref[...], w_ref[...],
                preferred_element_type=jnp.float32,
                precision=lax.Precision.HIGHEST)
    o_ref[...] = z * norm[:, None]


@jax.jit
def _tc_matmul_scale(x, w, degp):
    return pl.pallas_call(
        _mm_body,
        out_shape=jax.ShapeDtypeStruct((NPAD, D), jnp.float32),
        grid=(NPAD // BLK,),
        in_specs=[
            pl.BlockSpec((BLK, D), lambda i: (i, 0)),
            pl.BlockSpec((D, D), lambda i: (0, 0)),
            pl.BlockSpec((NW, BLK), lambda i: (0, i)),
        ],
        out_specs=pl.BlockSpec((BLK, D), lambda i: (i, 0)),
    )(x, w, degp)


def _post_body(p_ref, degp_ref, b_ref, o_ref):
    deg = jnp.sum(degp_ref[...], axis=0)
    norm = lax.rsqrt(jnp.maximum(deg, 1.0))
    agg = p_ref[0] + p_ref[1]
    o_ref[...] = agg * norm[:, None] + b_ref[...][None, :]


@jax.jit
def _tc_post(p, degp, b):
    return pl.pallas_call(
        _post_body,
        out_shape=jax.ShapeDtypeStruct((NPAD, D), jnp.float32),
        grid=(NPAD // BLK,),
        in_specs=[
            pl.BlockSpec((NC, BLK, D), lambda i: (0, i, 0)),
            pl.BlockSpec((NW, BLK), lambda i: (0, i)),
            pl.BlockSpec((D,), lambda i: (0,)),
        ],
        out_specs=pl.BlockSpec((BLK, D), lambda i: (i, 0)),
    )(p, degp, b)


def kernel(ndata, edge_index, W, b):
    src = edge_index[0].astype(jnp.int32)
    dst = edge_index[1].astype(jnp.int32)
    x_pad = jnp.pad(ndata, ((0, NPAD - N), (0, 0)))
    npe = E_PAD - E
    fill_s = jnp.arange(npe, dtype=jnp.int32) % N
    fill_d = N + jnp.arange(npe, dtype=jnp.int32) % (NPAD - N)
    src_p = jnp.concatenate([src, fill_s])
    dst_p = jnp.concatenate([dst, fill_d])
    packed2d = (src_p | (dst_p << 16)).reshape(NCHP, CH)
    dego_p, degi_p = _sc_hist(src, dst)
    h = _tc_matmul_scale(x_pad, W, dego_p)
    p = _sc_agg(h, packed2d)
    return _tc_post(p, degi_p, b)[:N]


# speedup vs baseline: 12.0302x; 1.1057x over previous
import dataclasses

import jax
import jax.numpy as jnp
from jax import lax
from jax.experimental import pallas as pl
from jax.experimental.pallas import tpu as pltpu
from jax.experimental.pallas import tpu_sc as plsc

N = 10000
NPAD = 10240
E = 320000
D = 128

NC = 2
NS = 16
NW = NC * NS

CH = 128
E_PAD = 327680
NCHP = E_PAD // CH
CPT = NCHP // NW
NBUF = 2
ROWS_PER_TILE = NPAD // NS
EDGES_PER_TILE_HIST = E // NW

_mesh = plsc.VectorSubcoreMesh(core_axis_name="c", subcore_axis_name="s")

_sc_params = pltpu.CompilerParams()
if "needs_layout_passes" in pltpu.CompilerParams.__dataclass_fields__:
    _sc_params = dataclasses.replace(_sc_params, needs_layout_passes=False)


def _hist_body(ei_hbm, dego_hbm, degi_hbm, idx_s, idx_d, ho_v, hi_v, sem, semd):
    cid = lax.axis_index("c")
    sid = lax.axis_index("s")
    wid = sid * NC + cid

    zeros16 = jnp.zeros((16,), jnp.float32)
    ones16 = jnp.ones((16,), jnp.float32)

    base = wid * EDGES_PER_TILE_HIST
    pltpu.async_copy(ei_hbm.at[pl.ds(base, EDGES_PER_TILE_HIST)], idx_s, sem)
    pltpu.async_copy(ei_hbm.at[pl.ds(E + base, EDGES_PER_TILE_HIST)], idx_d, semd)

    @pl.loop(0, NPAD // 64)
    def _zero(i):
        for q in range(4):
            ho_v[pl.ds(i * 64 + q * 16, 16)] = zeros16
            hi_v[pl.ds(i * 64 + q * 16, 16)] = zeros16

    pltpu.make_async_copy(
        ei_hbm.at[pl.ds(base, EDGES_PER_TILE_HIST)], idx_s, sem).wait()

    @pl.loop(0, EDGES_PER_TILE_HIST // 16)
    def _accum_src(i):
        v = idx_s[pl.ds(i * 16, 16)]
        plsc.addupdate_scatter(ho_v, [v], ones16)

    pltpu.sync_copy(ho_v, dego_hbm.at[wid])

    pltpu.make_async_copy(
        ei_hbm.at[pl.ds(E + base, EDGES_PER_TILE_HIST)], idx_d, semd).wait()

    @pl.loop(0, EDGES_PER_TILE_HIST // 16)
    def _accum_dst(i):
        v = idx_d[pl.ds(i * 16, 16)]
        plsc.addupdate_scatter(hi_v, [v], ones16)

    pltpu.sync_copy(hi_v, degi_hbm.at[wid])


@jax.jit
def _sc_hist(edge_index):
    k = pl.kernel(
        _hist_body,
        out_type=(
            jax.ShapeDtypeStruct((NW, NPAD), jnp.float32),
            jax.ShapeDtypeStruct((NW, NPAD), jnp.float32),
        ),
        mesh=_mesh,
        scratch_types=[
            pltpu.VMEM((EDGES_PER_TILE_HIST,), jnp.int32),
            pltpu.VMEM((EDGES_PER_TILE_HIST,), jnp.int32),
            pltpu.VMEM((NPAD,), jnp.float32),
            pltpu.VMEM((NPAD,), jnp.float32),
            pltpu.SemaphoreType.DMA,
            pltpu.SemaphoreType.DMA,
        ],
        compiler_params=_sc_params,
    )
    return k(edge_index)


def _agg_body(h_hbm, pk_hbm, out_hbm,
              pkb, r0, r1, su0, su1, du0, du1, agg_sh,
              g0, g1, s0, s1):
    cid = lax.axis_index("c")
    sid = lax.axis_index("s")
    wid = sid * NC + cid

    rows = (r0, r1)
    src_u = (su0, su1)
    dst_u = (du0, du1)
    gsem = (g0, g1)
    ssem = (s0, s1)

    zeros16 = jnp.zeros((16,), jnp.float32)
    mask16 = jnp.full((16,), 0xFFFF, jnp.int32)
    sixteen = jnp.full((16,), 16, jnp.int32)

    def unpack(c, p):
        for k8 in range(CH // 16):
            v = pkb[c, pl.ds(k8 * 16, 16)]
            src_u[p][pl.ds(k8 * 16, 16)] = jnp.bitwise_and(v, mask16)
            dst_u[p][pl.ds(k8 * 16, 16)] = lax.shift_right_logical(v, sixteen)

    base = wid * CPT
    pltpu.sync_copy(pk_hbm.at[pl.ds(base, CPT)], pkb)

    @pl.loop(0, CH)
    def _zero(r):
        for k8 in range(D // 16):
            r0[r, pl.ds(k8 * 16, 16)] = zeros16

    row0 = sid * ROWS_PER_TILE

    @pl.loop(0, ROWS_PER_TILE // CH)
    def _zinit(j):
        pltpu.sync_copy(r0, agg_sh.at[pl.ds(row0 + j * CH, CH)])

    for p in range(NBUF):
        unpack(p, p)
        pltpu.async_copy(h_hbm.at[src_u[p]], rows[p], gsem[p])

    plsc.subcore_barrier()

    @pl.loop(0, CPT // 2 - 1)
    def _unit(j):
        u = j * 2
        for p in range(NBUF):
            pltpu.make_async_copy(h_hbm.at[src_u[p]], rows[p], gsem[p]).wait()
            pltpu.async_copy(rows[p], agg_sh.at[dst_u[p]], ssem[p], add=True)
        for p in range(NBUF):
            c = u + p
            pltpu.make_async_copy(rows[p], agg_sh.at[dst_u[p]], ssem[p]).wait()
            unpack(c + NBUF, p)
            pltpu.async_copy(h_hbm.at[src_u[p]], rows[p], gsem[p])

    for p in range(NBUF):
        pltpu.make_async_copy(h_hbm.at[src_u[p]], rows[p], gsem[p]).wait()
        pltpu.async_copy(rows[p], agg_sh.at[dst_u[p]], ssem[p], add=True)
    for p in range(NBUF):
        pltpu.make_async_copy(rows[p], agg_sh.at[dst_u[p]], ssem[p]).wait()

    plsc.subcore_barrier()

    pltpu.sync_copy(agg_sh.at[pl.ds(row0, ROWS_PER_TILE)],
                    out_hbm.at[cid, pl.ds(row0, ROWS_PER_TILE)])


@jax.jit
def _sc_agg(h, packed2d):
    k = pl.kernel(
        _agg_body,
        out_type=jax.ShapeDtypeStruct((NC, NPAD, D), jnp.float32),
        mesh=_mesh,
        scratch_types=[
            pltpu.VMEM((CPT, CH), jnp.int32),
            pltpu.VMEM((CH, D), jnp.float32),
            pltpu.VMEM((CH, D), jnp.float32),
            pltpu.VMEM((CH,), jnp.int32),
            pltpu.VMEM((CH,), jnp.int32),
            pltpu.VMEM((CH,), jnp.int32),
            pltpu.VMEM((CH,), jnp.int32),
            pltpu.VMEM_SHARED((NPAD, D), jnp.float32),
            pltpu.SemaphoreType.DMA,
            pltpu.SemaphoreType.DMA,
            pltpu.SemaphoreType.DMA,
            pltpu.SemaphoreType.DMA,
        ],
        compiler_params=_sc_params,
    )
    return k(h, packed2d)


BLK = 2048


def _mm_body(x_ref, w_ref, degp_ref, o_ref):
    deg = jnp.sum(degp_ref[...], axis=0)
    norm = lax.rsqrt(jnp.maximum(deg, 1.0))
    z = jnp.dot(x_ref[...], w_ref[...],
                preferred_element_type=jnp.float32,
                precision=lax.Precision.HIGHEST)
    o_ref[...] = z * norm[:, None]


@jax.jit
def _tc_matmul_scale(x, w, degp):
    return pl.pallas_call(
        _mm_body,
        out_shape=jax.ShapeDtypeStruct((NPAD, D), jnp.float32),
        grid=(NPAD // BLK,),
        in_specs=[
            pl.BlockSpec((BLK, D), lambda i: (i, 0)),
            pl.BlockSpec((D, D), lambda i: (0, 0)),
            pl.BlockSpec((NW, BLK), lambda i: (0, i)),
        ],
        out_specs=pl.BlockSpec((BLK, D), lambda i: (i, 0)),
    )(x, w, degp)


def _post_body(p_ref, degp_ref, b_ref, o_ref):
    deg = jnp.sum(degp_ref[...], axis=0)
    norm = lax.rsqrt(jnp.maximum(deg, 1.0))
    agg = p_ref[0] + p_ref[1]
    o_ref[...] = agg * norm[:, None] + b_ref[...][None, :]


@jax.jit
def _tc_post(p, degp, b):
    return pl.pallas_call(
        _post_body,
        out_shape=jax.ShapeDtypeStruct((NPAD, D), jnp.float32),
        grid=(NPAD // BLK,),
        in_specs=[
            pl.BlockSpec((NC, BLK, D), lambda i: (0, i, 0)),
            pl.BlockSpec((NW, BLK), lambda i: (0, i)),
            pl.BlockSpec((D,), lambda i: (0,)),
        ],
        out_specs=pl.BlockSpec((BLK, D), lambda i: (i, 0)),
    )(p, degp, b)


def kernel(ndata, edge_index, W, b):
    src = edge_index[0].astype(jnp.int32)
    dst = edge_index[1].astype(jnp.int32)
    x_pad = jnp.pad(ndata, ((0, NPAD - N), (0, 0)))
    npe = E_PAD - E
    fill_s = jnp.arange(npe, dtype=jnp.int32) % N
    fill_d = N + jnp.arange(npe, dtype=jnp.int32) % (NPAD - N)
    src_p = jnp.concatenate([src, fill_s])
    dst_p = jnp.concatenate([dst, fill_d])
    packed2d = (src_p | (dst_p << 16)).reshape(NCHP, CH)
    dego_p, degi_p = _sc_hist(jnp.ravel(edge_index.astype(jnp.int32)))
    h = _tc_matmul_scale(x_pad, W, dego_p)
    p = _sc_agg(h, packed2d)
    return _tc_post(p, degi_p, b)[:N]


# speedup vs baseline: 13.4681x; 1.1195x over previous
import dataclasses

import jax
import jax.numpy as jnp
from jax import lax
from jax.experimental import pallas as pl
from jax.experimental.pallas import tpu as pltpu
from jax.experimental.pallas import tpu_sc as plsc

N = 10000
NPAD = 10240
E = 320000
D = 128

NC = 2
NS = 16
NW = NC * NS

CH = 80
NCHP = E // CH
CPT = NCHP // NW
NBUF = 3
MAIN_UNITS = (CPT - NBUF) // NBUF
ROWS_PER_TILE = NPAD // NS
EDGES_PER_TILE_HIST = E // NW

_mesh = plsc.VectorSubcoreMesh(core_axis_name="c", subcore_axis_name="s")

_sc_params = pltpu.CompilerParams()
if "needs_layout_passes" in pltpu.CompilerParams.__dataclass_fields__:
    _sc_params = dataclasses.replace(_sc_params, needs_layout_passes=False)


def _hist_body(ei_hbm, dego_hbm, degi_hbm, idx_s, idx_d, ho_v, hi_v, sem, semd):
    cid = lax.axis_index("c")
    sid = lax.axis_index("s")
    wid = sid * NC + cid

    zeros16 = jnp.zeros((16,), jnp.float32)
    ones16 = jnp.ones((16,), jnp.float32)

    base = wid * EDGES_PER_TILE_HIST
    pltpu.async_copy(ei_hbm.at[pl.ds(base, EDGES_PER_TILE_HIST)], idx_s, sem)
    pltpu.async_copy(ei_hbm.at[pl.ds(E + base, EDGES_PER_TILE_HIST)], idx_d, semd)

    @pl.loop(0, NPAD // 64)
    def _zero(i):
        for q in range(4):
            ho_v[pl.ds(i * 64 + q * 16, 16)] = zeros16
            hi_v[pl.ds(i * 64 + q * 16, 16)] = zeros16

    pltpu.make_async_copy(
        ei_hbm.at[pl.ds(base, EDGES_PER_TILE_HIST)], idx_s, sem).wait()

    @pl.loop(0, EDGES_PER_TILE_HIST // 16)
    def _accum_src(i):
        v = idx_s[pl.ds(i * 16, 16)]
        plsc.addupdate_scatter(ho_v, [v], ones16)

    pltpu.sync_copy(ho_v, dego_hbm.at[wid])

    pltpu.make_async_copy(
        ei_hbm.at[pl.ds(E + base, EDGES_PER_TILE_HIST)], idx_d, semd).wait()

    @pl.loop(0, EDGES_PER_TILE_HIST // 16)
    def _accum_dst(i):
        v = idx_d[pl.ds(i * 16, 16)]
        plsc.addupdate_scatter(hi_v, [v], ones16)

    pltpu.sync_copy(hi_v, degi_hbm.at[wid])


@jax.jit
def _sc_hist(edge_index):
    k = pl.kernel(
        _hist_body,
        out_type=(
            jax.ShapeDtypeStruct((NW, NPAD), jnp.float32),
            jax.ShapeDtypeStruct((NW, NPAD), jnp.float32),
        ),
        mesh=_mesh,
        scratch_types=[
            pltpu.VMEM((EDGES_PER_TILE_HIST,), jnp.int32),
            pltpu.VMEM((EDGES_PER_TILE_HIST,), jnp.int32),
            pltpu.VMEM((NPAD,), jnp.float32),
            pltpu.VMEM((NPAD,), jnp.float32),
            pltpu.SemaphoreType.DMA,
            pltpu.SemaphoreType.DMA,
        ],
        compiler_params=_sc_params,
    )
    return k(edge_index)


def _agg_body(h_hbm, pk_hbm, out_hbm,
              pkb, r0, r1, r2, su0, su1, su2, du0, du1, du2, agg_sh,
              g0, g1, g2, s0, s1, s2):
    cid = lax.axis_index("c")
    sid = lax.axis_index("s")
    wid = sid * NC + cid

    rows = (r0, r1, r2)
    src_u = (su0, su1, su2)
    dst_u = (du0, du1, du2)
    gsem = (g0, g1, g2)
    ssem = (s0, s1, s2)

    zeros16 = jnp.zeros((16,), jnp.float32)
    mask16 = jnp.full((16,), 0xFFFF, jnp.int32)
    sixteen = jnp.full((16,), 16, jnp.int32)

    def unpack(c, p):
        for k8 in range(CH // 16):
            v = pkb[c, pl.ds(k8 * 16, 16)]
            src_u[p][pl.ds(k8 * 16, 16)] = jnp.bitwise_and(v, mask16)
            dst_u[p][pl.ds(k8 * 16, 16)] = lax.shift_right_logical(v, sixteen)

    pltpu.sync_copy(pk_hbm.at[wid], pkb)

    @pl.loop(0, CH)
    def _zero(r):
        for k8 in range(D // 16):
            r0[r, pl.ds(k8 * 16, 16)] = zeros16

    row0 = sid * ROWS_PER_TILE

    @pl.loop(0, ROWS_PER_TILE // CH)
    def _zinit(j):
        pltpu.sync_copy(r0, agg_sh.at[pl.ds(row0 + j * CH, CH)])

    for p in range(NBUF):
        unpack(p, p)
        pltpu.async_copy(h_hbm.at[src_u[p]], rows[p], gsem[p])

    plsc.subcore_barrier()

    def wait_gather(p):
        pltpu.make_async_copy(h_hbm.at[src_u[p]], rows[p], gsem[p]).wait()

    def start_scatter(p):
        pltpu.async_copy(rows[p], agg_sh.at[dst_u[p]], ssem[p], add=True)

    def wait_scatter(p):
        pltpu.make_async_copy(rows[p], agg_sh.at[dst_u[p]], ssem[p]).wait()

    def start_gather(c, p):
        unpack(c, p)
        pltpu.async_copy(h_hbm.at[src_u[p]], rows[p], gsem[p])

    @pl.loop(0, MAIN_UNITS)
    def _unit(j):
        u = j * NBUF
        for p in range(NBUF):
            wait_gather(p)
            start_scatter(p)
        for p in range(NBUF):
            wait_scatter(p)
            start_gather(u + NBUF + p, p)

    tail0 = MAIN_UNITS * NBUF
    ntail = CPT - tail0 - NBUF
    for p in range(NBUF):
        wait_gather(p)
        start_scatter(p)
    for p in range(ntail):
        wait_scatter(p)
        start_gather(tail0 + NBUF + p, p)
    for p in range(ntail):
        wait_gather(p)
        start_scatter(p)
    for p in range(ntail):
        wait_scatter(p)
    for p in range(ntail, NBUF):
        wait_scatter(p)

    plsc.subcore_barrier()

    pltpu.sync_copy(agg_sh.at[pl.ds(row0, ROWS_PER_TILE)],
                    out_hbm.at[cid, pl.ds(row0, ROWS_PER_TILE)])


@jax.jit
def _sc_agg(h, packed2d):
    k = pl.kernel(
        _agg_body,
        out_type=jax.ShapeDtypeStruct((NC, NPAD, D), jnp.float32),
        mesh=_mesh,
        scratch_types=[
            pltpu.VMEM((CPT, CH), jnp.int32),
            pltpu.VMEM((CH, D), jnp.float32),
            pltpu.VMEM((CH, D), jnp.float32),
            pltpu.VMEM((CH, D), jnp.float32),
            pltpu.VMEM((CH,), jnp.int32),
            pltpu.VMEM((CH,), jnp.int32),
            pltpu.VMEM((CH,), jnp.int32),
            pltpu.VMEM((CH,), jnp.int32),
            pltpu.VMEM((CH,), jnp.int32),
            pltpu.VMEM((CH,), jnp.int32),
            pltpu.VMEM_SHARED((NPAD, D), jnp.float32),
            pltpu.SemaphoreType.DMA,
            pltpu.SemaphoreType.DMA,
            pltpu.SemaphoreType.DMA,
            pltpu.SemaphoreType.DMA,
            pltpu.SemaphoreType.DMA,
            pltpu.SemaphoreType.DMA,
        ],
        compiler_params=_sc_params,
    )
    return k(h, packed2d)


BLK = 2048


def _mm_body(x_ref, w_ref, degp_ref, o_ref):
    deg = jnp.sum(degp_ref[...], axis=0)
    norm = lax.rsqrt(jnp.maximum(deg, 1.0))
    z = jnp.dot(x_ref[...], w_ref[...],
                preferred_element_type=jnp.float32,
                precision=lax.Precision.HIGHEST)
    o_ref[...] = z * norm[:, None]


@jax.jit
def _tc_matmul_scale(x, w, degp):
    return pl.pallas_call(
        _mm_body,
        out_shape=jax.ShapeDtypeStruct((NPAD, D), jnp.float32),
        grid=(NPAD // BLK,),
        in_specs=[
            pl.BlockSpec((BLK, D), lambda i: (i, 0)),
            pl.BlockSpec((D, D), lambda i: (0, 0)),
            pl.BlockSpec((NW, BLK), lambda i: (0, i)),
        ],
        out_specs=pl.BlockSpec((BLK, D), lambda i: (i, 0)),
    )(x, w, degp)


def _post_body(p_ref, degp_ref, b_ref, o_ref):
    deg = jnp.sum(degp_ref[...], axis=0)
    norm = lax.rsqrt(jnp.maximum(deg, 1.0))
    agg = p_ref[0] + p_ref[1]
    o_ref[...] = agg * norm[:, None] + b_ref[...][None, :]


@jax.jit
def _tc_post(p, degp, b):
    return pl.pallas_call(
        _post_body,
        out_shape=jax.ShapeDtypeStruct((NPAD, D), jnp.float32),
        grid=(NPAD // BLK,),
        in_specs=[
            pl.BlockSpec((NC, BLK, D), lambda i: (0, i, 0)),
            pl.BlockSpec((NW, BLK), lambda i: (0, i)),
            pl.BlockSpec((D,), lambda i: (0,)),
        ],
        out_specs=pl.BlockSpec((BLK, D), lambda i: (i, 0)),
    )(p, degp, b)


def kernel(ndata, edge_index, W, b):
    src = edge_index[0].astype(jnp.int32)
    dst = edge_index[1].astype(jnp.int32)
    x_pad = jnp.pad(ndata, ((0, NPAD - N), (0, 0)))
    packed2d = (src | (dst << 16)).reshape(NW, CPT, CH)
    dego_p, degi_p = _sc_hist(jnp.ravel(edge_index.astype(jnp.int32)))
    h = _tc_matmul_scale(x_pad, W, dego_p)
    p = _sc_agg(h, packed2d)
    return _tc_post(p, degi_p, b)[:N]


# speedup vs baseline: 14.8487x; 1.1025x over previous
import dataclasses

import jax
import jax.numpy as jnp
from jax import lax
from jax.experimental import pallas as pl
from jax.experimental.pallas import tpu as pltpu
from jax.experimental.pallas import tpu_sc as plsc

N = 10000
NPAD = 10240
E = 320000
D = 128

NC = 2
NS = 16
NW = NC * NS

CH = 80
NCHP = E // CH
CPT = NCHP // NW
NBUF = 3
MAIN_UNITS = (CPT - NBUF) // NBUF
ROWS_PER_TILE = NPAD // NS
EPT = E // NW

_mesh = plsc.VectorSubcoreMesh(core_axis_name="c", subcore_axis_name="s")

_sc_params = pltpu.CompilerParams()
if "needs_layout_passes" in pltpu.CompilerParams.__dataclass_fields__:
    _sc_params = dataclasses.replace(_sc_params, needs_layout_passes=False)


def _hist_body(ei_hbm, dego_hbm, degi_hbm, pk_hbm,
               idx_s, idx_d, pk_v, ho_v, hi_v, sem, semd):
    cid = lax.axis_index("c")
    sid = lax.axis_index("s")
    wid = sid * NC + cid

    zeros16 = jnp.zeros((16,), jnp.float32)
    ones16 = jnp.ones((16,), jnp.float32)
    sixteen = jnp.full((16,), 16, jnp.int32)

    base = wid * EPT
    pltpu.async_copy(ei_hbm.at[pl.ds(base, EPT)], idx_s, sem)
    pltpu.async_copy(ei_hbm.at[pl.ds(E + base, EPT)], idx_d, semd)

    @pl.loop(0, NPAD // 64)
    def _zero(i):
        for q in range(4):
            ho_v[pl.ds(i * 64 + q * 16, 16)] = zeros16
            hi_v[pl.ds(i * 64 + q * 16, 16)] = zeros16

    pltpu.make_async_copy(ei_hbm.at[pl.ds(base, EPT)], idx_s, sem).wait()
    pltpu.make_async_copy(ei_hbm.at[pl.ds(E + base, EPT)], idx_d, semd).wait()

    @pl.loop(0, EPT // 16)
    def _accum(i):
        s = idx_s[pl.ds(i * 16, 16)]
        d = idx_d[pl.ds(i * 16, 16)]
        plsc.addupdate_scatter(ho_v, [s], ones16)
        plsc.addupdate_scatter(hi_v, [d], ones16)
        pk_v[pl.ds(i * 16, 16)] = jnp.bitwise_or(s, lax.shift_left(d, sixteen))

    pltpu.sync_copy(pk_v, pk_hbm.at[wid])
    pltpu.sync_copy(ho_v, dego_hbm.at[wid])
    pltpu.sync_copy(hi_v, degi_hbm.at[wid])


@jax.jit
def _sc_hist(ei_flat):
    k = pl.kernel(
        _hist_body,
        out_type=(
            jax.ShapeDtypeStruct((NW, NPAD), jnp.float32),
            jax.ShapeDtypeStruct((NW, NPAD), jnp.float32),
            jax.ShapeDtypeStruct((NW, EPT), jnp.int32),
        ),
        mesh=_mesh,
        scratch_types=[
            pltpu.VMEM((EPT,), jnp.int32),
            pltpu.VMEM((EPT,), jnp.int32),
            pltpu.VMEM((EPT,), jnp.int32),
            pltpu.VMEM((NPAD,), jnp.float32),
            pltpu.VMEM((NPAD,), jnp.float32),
            pltpu.SemaphoreType.DMA,
            pltpu.SemaphoreType.DMA,
        ],
        compiler_params=_sc_params,
    )
    return k(ei_flat)


def _agg_body(h_hbm, pk_hbm, out_hbm,
              pkb, r0, r1, r2, su0, su1, su2, du0, du1, du2, agg_sh,
              g0, g1, g2, s0, s1, s2):
    cid = lax.axis_index("c")
    sid = lax.axis_index("s")
    wid = sid * NC + cid

    rows = (r0, r1, r2)
    src_u = (su0, su1, su2)
    dst_u = (du0, du1, du2)
    gsem = (g0, g1, g2)
    ssem = (s0, s1, s2)

    zeros16 = jnp.zeros((16,), jnp.float32)
    mask16 = jnp.full((16,), 0xFFFF, jnp.int32)
    sixteen = jnp.full((16,), 16, jnp.int32)

    def unpack(c, p):
        for k8 in range(CH // 16):
            v = pkb[pl.ds(c * CH + k8 * 16, 16)]
            src_u[p][pl.ds(k8 * 16, 16)] = jnp.bitwise_and(v, mask16)
            dst_u[p][pl.ds(k8 * 16, 16)] = lax.shift_right_logical(v, sixteen)

    pltpu.sync_copy(pk_hbm.at[wid], pkb)

    @pl.loop(0, CH)
    def _zero(r):
        for k8 in range(D // 16):
            r0[r, pl.ds(k8 * 16, 16)] = zeros16

    row0 = sid * ROWS_PER_TILE

    @pl.loop(0, ROWS_PER_TILE // CH)
    def _zinit(j):
        pltpu.sync_copy(r0, agg_sh.at[pl.ds(row0 + j * CH, CH)])

    for p in range(NBUF):
        unpack(p, p)
        pltpu.async_copy(h_hbm.at[src_u[p]], rows[p], gsem[p])

    plsc.subcore_barrier()

    def wait_gather(p):
        pltpu.make_async_copy(h_hbm.at[src_u[p]], rows[p], gsem[p]).wait()

    def start_scatter(p):
        pltpu.async_copy(rows[p], agg_sh.at[dst_u[p]], ssem[p], add=True)

    def wait_scatter(p):
        pltpu.make_async_copy(rows[p], agg_sh.at[dst_u[p]], ssem[p]).wait()

    def start_gather(c, p):
        unpack(c, p)
        pltpu.async_copy(h_hbm.at[src_u[p]], rows[p], gsem[p])

    @pl.loop(0, MAIN_UNITS)
    def _unit(j):
        u = j * NBUF
        for p in range(NBUF):
            wait_gather(p)
            start_scatter(p)
        for p in range(NBUF):
            wait_scatter(p)
            start_gather(u + NBUF + p, p)

    tail0 = MAIN_UNITS * NBUF
    ntail = CPT - tail0 - NBUF
    for p in range(NBUF):
        wait_gather(p)
        start_scatter(p)
    for p in range(ntail):
        wait_scatter(p)
        start_gather(tail0 + NBUF + p, p)
    for p in range(ntail):
        wait_gather(p)
        start_scatter(p)
    for p in range(ntail):
        wait_scatter(p)
    for p in range(ntail, NBUF):
        wait_scatter(p)

    plsc.subcore_barrier()

    pltpu.sync_copy(agg_sh.at[pl.ds(row0, ROWS_PER_TILE)],
                    out_hbm.at[cid, pl.ds(row0, ROWS_PER_TILE)])


@jax.jit
def _sc_agg(h, packed):
    k = pl.kernel(
        _agg_body,
        out_type=jax.ShapeDtypeStruct((NC, NPAD, D), jnp.float32),
        mesh=_mesh,
        scratch_types=[
            pltpu.VMEM((EPT,), jnp.int32),
            pltpu.VMEM((CH, D), jnp.float32),
            pltpu.VMEM((CH, D), jnp.float32),
            pltpu.VMEM((CH, D), jnp.float32),
            pltpu.VMEM((CH,), jnp.int32),
            pltpu.VMEM((CH,), jnp.int32),
            pltpu.VMEM((CH,), jnp.int32),
            pltpu.VMEM((CH,), jnp.int32),
            pltpu.VMEM((CH,), jnp.int32),
            pltpu.VMEM((CH,), jnp.int32),
            pltpu.VMEM_SHARED((NPAD, D), jnp.float32),
            pltpu.SemaphoreType.DMA,
            pltpu.SemaphoreType.DMA,
            pltpu.SemaphoreType.DMA,
            pltpu.SemaphoreType.DMA,
            pltpu.SemaphoreType.DMA,
            pltpu.SemaphoreType.DMA,
        ],
        compiler_params=_sc_params,
    )
    return k(h, packed)


BLK = 2048


def _mm_body(x_ref, w_ref, degp_ref, o_ref):
    deg = jnp.sum(degp_ref[...], axis=0)
    norm = lax.rsqrt(jnp.maximum(deg, 1.0))
    z = jnp.dot(x_ref[...], w_ref[...],
                preferred_element_type=jnp.float32,
                precision=lax.Precision.HIGHEST)
    o_ref[...] = z * norm[:, None]


@jax.jit
def _tc_matmul_scale(x, w, degp):
    return pl.pallas_call(
        _mm_body,
        out_shape=jax.ShapeDtypeStruct((NPAD, D), jnp.float32),
        grid=(NPAD // BLK,),
        in_specs=[
            pl.BlockSpec((BLK, D), lambda i: (i, 0)),
            pl.BlockSpec((D, D), lambda i: (0, 0)),
            pl.BlockSpec((NW, BLK), lambda i: (0, i)),
        ],
        out_specs=pl.BlockSpec((BLK, D), lambda i: (i, 0)),
    )(x, w, degp)


def _post_body(p_ref, degp_ref, b_ref, o_ref):
    deg = jnp.sum(degp_ref[...], axis=0)
    norm = lax.rsqrt(jnp.maximum(deg, 1.0))
    agg = p_ref[0] + p_ref[1]
    o_ref[...] = agg * norm[:, None] + b_ref[...][None, :]


@jax.jit
def _tc_post(p, degp, b):
    return pl.pallas_call(
        _post_body,
        out_shape=jax.ShapeDtypeStruct((N, D), jnp.float32),
        grid=(NPAD // BLK,),
        in_specs=[
            pl.BlockSpec((NC, BLK, D), lambda i: (0, i, 0)),
            pl.BlockSpec((NW, BLK), lambda i: (0, i)),
            pl.BlockSpec((D,), lambda i: (0,)),
        ],
        out_specs=pl.BlockSpec((BLK, D), lambda i: (i, 0)),
    )(p, degp, b)


def kernel(ndata, edge_index, W, b):
    ei_flat = jnp.ravel(edge_index.astype(jnp.int32))
    x_pad = jnp.pad(ndata, ((0, NPAD - N), (0, 0)))
    dego_p, degi_p, packed = _sc_hist(ei_flat)
    h = _tc_matmul_scale(x_pad, W, dego_p)
    p = _sc_agg(h, packed)
    return _tc_post(p, degi_p, b)


# speedup vs baseline: 15.8877x; 1.0700x over previous
import dataclasses

import jax
import jax.numpy as jnp
from jax import lax
from jax.experimental import pallas as pl
from jax.experimental.pallas import tpu as pltpu
from jax.experimental.pallas import tpu_sc as plsc

N = 10000
NPAD = 10240
E = 320000
D = 128

NC = 2
NS = 16
NW = NC * NS

EPT = 9984
EPT_LAST = E - EPT * (NW - 1)
CH = 64
CPT = EPT // CH
CPT_LAST = EPT_LAST // CH
NBUF = 4
ROWS_PER_TILE = NPAD // NS

_mesh = plsc.VectorSubcoreMesh(core_axis_name="c", subcore_axis_name="s")

_sc_params = pltpu.CompilerParams()
if "needs_layout_passes" in pltpu.CompilerParams.__dataclass_fields__:
    _sc_params = dataclasses.replace(_sc_params, needs_layout_passes=False)


def _hist_body(ei_hbm, dego_hbm, degi_hbm, pk_hbm,
               idx_sd, pk_v, ho_v, hi_v, sem):
    cid = lax.axis_index("c")
    sid = lax.axis_index("s")
    wid = sid * NC + cid

    zeros16 = jnp.zeros((16,), jnp.float32)
    ones16 = jnp.ones((16,), jnp.float32)
    sixteen = jnp.full((16,), 16, jnp.int32)

    base = wid * EPT
    pltpu.async_copy(ei_hbm.at[:, pl.ds(base, EPT_LAST)], idx_sd, sem)

    @pl.loop(0, NPAD // 64)
    def _zero(i):
        for q in range(4):
            ho_v[pl.ds(i * 64 + q * 16, 16)] = zeros16
            hi_v[pl.ds(i * 64 + q * 16, 16)] = zeros16

    pltpu.make_async_copy(ei_hbm.at[:, pl.ds(base, EPT_LAST)], idx_sd, sem).wait()

    nvec = jnp.where(wid == NW - 1, EPT_LAST // 16, EPT // 16)

    @pl.loop(0, nvec)
    def _accum(i):
        s = idx_sd[0, pl.ds(i * 16, 16)]
        d = idx_sd[1, pl.ds(i * 16, 16)]
        plsc.addupdate_scatter(ho_v, [s], ones16)
        plsc.addupdate_scatter(hi_v, [d], ones16)
        pk_v[pl.ds(i * 16, 16)] = jnp.bitwise_or(s, lax.shift_left(d, sixteen))

    pltpu.sync_copy(pk_v, pk_hbm.at[wid])
    pltpu.sync_copy(ho_v, dego_hbm.at[wid])
    pltpu.sync_copy(hi_v, degi_hbm.at[wid])


@jax.jit
def _sc_hist(ei):
    k = pl.kernel(
        _hist_body,
        out_type=(
            jax.ShapeDtypeStruct((NW, NPAD), jnp.float32),
            jax.ShapeDtypeStruct((NW, NPAD), jnp.float32),
            jax.ShapeDtypeStruct((NW, EPT_LAST), jnp.int32),
        ),
        mesh=_mesh,
        scratch_types=[
            pltpu.VMEM((2, EPT_LAST), jnp.int32),
            pltpu.VMEM((EPT_LAST,), jnp.int32),
            pltpu.VMEM((NPAD,), jnp.float32),
            pltpu.VMEM((NPAD,), jnp.float32),
            pltpu.SemaphoreType.DMA,
        ],
        compiler_params=_sc_params,
    )
    return k(ei)


def _agg_body(h_hbm, pk_hbm, out_hbm,
              pkb, r0, r1, r2, r3, su0, su1, su2, su3, du0, du1, du2, du3,
              agg_sh, g0, g1, g2, g3, s0, s1, s2, s3):
    cid = lax.axis_index("c")
    sid = lax.axis_index("s")
    wid = sid * NC + cid

    rows = (r0, r1, r2, r3)
    src_u = (su0, su1, su2, su3)
    dst_u = (du0, du1, du2, du3)
    gsem = (g0, g1, g2, g3)
    ssem = (s0, s1, s2, s3)

    zeros16 = jnp.zeros((16,), jnp.float32)
    mask16 = jnp.full((16,), 0xFFFF, jnp.int32)
    sixteen = jnp.full((16,), 16, jnp.int32)

    def unpack(c, p):
        for k8 in range(CH // 16):
            v = pkb[pl.ds(c * CH + k8 * 16, 16)]
            src_u[p][pl.ds(k8 * 16, 16)] = jnp.bitwise_and(v, mask16)
            dst_u[p][pl.ds(k8 * 16, 16)] = lax.shift_right_logical(v, sixteen)

    pltpu.sync_copy(pk_hbm.at[wid], pkb)

    @pl.loop(0, CH)
    def _zero(r):
        for k8 in range(D // 16):
            r0[r, pl.ds(k8 * 16, 16)] = zeros16

    row0 = sid * ROWS_PER_TILE

    @pl.loop(0, ROWS_PER_TILE // CH)
    def _zinit(j):
        pltpu.sync_copy(r0, agg_sh.at[pl.ds(row0 + j * CH, CH)])

    for p in range(NBUF):
        unpack(p, p)
        pltpu.async_copy(h_hbm.at[src_u[p]], rows[p], gsem[p])

    plsc.subcore_barrier()

    def wait_gather(p):
        pltpu.make_async_copy(h_hbm.at[src_u[p]], rows[p], gsem[p]).wait()

    def start_scatter(p):
        pltpu.async_copy(rows[p], agg_sh.at[dst_u[p]], ssem[p], add=True)

    def wait_scatter(p):
        pltpu.make_async_copy(rows[p], agg_sh.at[dst_u[p]], ssem[p]).wait()

    def start_gather(c, p):
        unpack(c, p)
        pltpu.async_copy(h_hbm.at[src_u[p]], rows[p], gsem[p])

    units = jnp.where(wid == NW - 1, CPT_LAST // NBUF, CPT // NBUF)

    @pl.loop(0, units - 1)
    def _unit(j):
        u = j * NBUF
        for p in range(NBUF):
            wait_gather(p)
            start_scatter(p)
        for p in range(NBUF):
            wait_scatter(p)
            start_gather(u + NBUF + p, p)

    for p in range(NBUF):
        wait_gather(p)
        start_scatter(p)
    for p in range(NBUF):
        wait_scatter(p)

    plsc.subcore_barrier()

    pltpu.sync_copy(agg_sh.at[pl.ds(row0, ROWS_PER_TILE)],
                    out_hbm.at[cid, pl.ds(row0, ROWS_PER_TILE)])


@jax.jit
def _sc_agg(h, packed):
    k = pl.kernel(
        _agg_body,
        out_type=jax.ShapeDtypeStruct((NC, NPAD, D), jnp.float32),
        mesh=_mesh,
        scratch_types=[
            pltpu.VMEM((EPT_LAST,), jnp.int32),
            pltpu.VMEM((CH, D), jnp.float32),
            pltpu.VMEM((CH, D), jnp.float32),
            pltpu.VMEM((CH, D), jnp.float32),
            pltpu.VMEM((CH, D), jnp.float32),
            pltpu.VMEM((CH,), jnp.int32),
            pltpu.VMEM((CH,), jnp.int32),
            pltpu.VMEM((CH,), jnp.int32),
            pltpu.VMEM((CH,), jnp.int32),
            pltpu.VMEM((CH,), jnp.int32),
            pltpu.VMEM((CH,), jnp.int32),
            pltpu.VMEM((CH,), jnp.int32),
            pltpu.VMEM((CH,), jnp.int32),
            pltpu.VMEM_SHARED((NPAD, D), jnp.float32),
            pltpu.SemaphoreType.DMA,
            pltpu.SemaphoreType.DMA,
            pltpu.SemaphoreType.DMA,
            pltpu.SemaphoreType.DMA,
            pltpu.SemaphoreType.DMA,
            pltpu.SemaphoreType.DMA,
            pltpu.SemaphoreType.DMA,
            pltpu.SemaphoreType.DMA,
        ],
        compiler_params=_sc_params,
    )
    return k(h, packed)


BLK = 2048


def _mm_body(x_ref, w_ref, degp_ref, o_ref):
    deg = jnp.sum(degp_ref[...], axis=0)
    norm = lax.rsqrt(jnp.maximum(deg, 1.0))
    z = jnp.dot(x_ref[...], w_ref[...],
                preferred_element_type=jnp.float32,
                precision=lax.Precision.HIGHEST)
    o_ref[...] = z * norm[:, None]


@jax.jit
def _tc_matmul_scale(x, w, degp):
    return pl.pallas_call(
        _mm_body,
        out_shape=jax.ShapeDtypeStruct((NPAD, D), jnp.float32),
        grid=(NPAD // BLK,),
        in_specs=[
            pl.BlockSpec((BLK, D), lambda i: (i, 0)),
            pl.BlockSpec((D, D), lambda i: (0, 0)),
            pl.BlockSpec((NW, BLK), lambda i: (0, i)),
        ],
        out_specs=pl.BlockSpec((BLK, D), lambda i: (i, 0)),
    )(x, w, degp)


def _post_body(p_ref, degp_ref, b_ref, o_ref):
    deg = jnp.sum(degp_ref[...], axis=0)
    norm = lax.rsqrt(jnp.maximum(deg, 1.0))
    agg = p_ref[0] + p_ref[1]
    o_ref[...] = agg * norm[:, None] + b_ref[...][None, :]


@jax.jit
def _tc_post(p, degp, b):
    return pl.pallas_call(
        _post_body,
        out_shape=jax.ShapeDtypeStruct((N, D), jnp.float32),
        grid=(NPAD // BLK,),
        in_specs=[
            pl.BlockSpec((NC, BLK, D), lambda i: (0, i, 0)),
            pl.BlockSpec((NW, BLK), lambda i: (0, i)),
            pl.BlockSpec((D,), lambda i: (0,)),
        ],
        out_specs=pl.BlockSpec((BLK, D), lambda i: (i, 0)),
    )(p, degp, b)


def kernel(ndata, edge_index, W, b):
    ei = edge_index.astype(jnp.int32)
    dego_p, degi_p, packed = _sc_hist(ei)
    h = _tc_matmul_scale(ndata, W, dego_p)
    p = _sc_agg(h, packed)
    return _tc_post(p, degi_p, b)
